# baseline (speedup 1.0000x reference)
import jax
import jax.numpy as jnp
from jax.experimental import pallas as pl
from jax.experimental.pallas import tpu as pltpu

_BM = 400


def _gcn_body(x_ref, w_ref, b_ref, adj_hbm, out_ref, adj_out_hbm,
              hidden_ref, buf, in_sems, out_sems):
    i = pl.program_id(0)
    nsteps = pl.num_programs(0)
    cur = jax.lax.rem(i, 2)
    nxt = jax.lax.rem(i + 1, 2)

    def in_copy(step, slot):
        return pltpu.make_async_copy(
            adj_hbm.at[pl.ds(step * _BM, _BM), :],
            buf.at[slot],
            in_sems.at[slot],
        )

    def out_copy(step, slot):
        return pltpu.make_async_copy(
            buf.at[slot],
            adj_out_hbm.at[pl.ds(step * _BM, _BM), :],
            out_sems.at[slot],
        )

    @pl.when(i == 0)
    def _start_and_hidden():
        in_copy(0, 0).start()
        hidden_ref[...] = (
            jax.lax.dot_general(
                x_ref[...], w_ref[...],
                dimension_numbers=(((1,), (1,)), ((), ())),
                preferred_element_type=jnp.float32,
            )
            + b_ref[...]
        ).astype(jnp.bfloat16)

    @pl.when(i + 1 < nsteps)
    def _prefetch_next():
        @pl.when(i >= 1)
        def _():
            out_copy(i - 1, nxt).wait()
        in_copy(i + 1, nxt).start()

    in_copy(i, cur).wait()
    out_copy(i, cur).start()
    out_ref[...] = jnp.maximum(
        jnp.dot(buf[cur].astype(jnp.bfloat16), hidden_ref[...],
                preferred_element_type=jnp.float32),
        0.0,
    )

    @pl.when(i == nsteps - 1)
    def _drain():
        @pl.when(i >= 1)
        def _():
            out_copy(i - 1, nxt).wait()
        out_copy(i, cur).wait()


def kernel(x, adj, W, b):
    n, d_in = x.shape
    d_out = W.shape[0]
    out, adj_out = pl.pallas_call(
        _gcn_body,
        grid=(n // _BM,),
        in_specs=[
            pl.BlockSpec((n, d_in), lambda i: (0, 0)),
            pl.BlockSpec((d_out, d_in), lambda i: (0, 0)),
            pl.BlockSpec((1, d_out), lambda i: (0, 0)),
            pl.BlockSpec(memory_space=pltpu.MemorySpace.HBM),
        ],
        out_specs=[
            pl.BlockSpec((_BM, d_out), lambda i: (i, 0)),
            pl.BlockSpec(memory_space=pltpu.MemorySpace.HBM),
        ],
        out_shape=[
            jax.ShapeDtypeStruct((n, d_out), jnp.float32),
            jax.ShapeDtypeStruct((n, n), jnp.float32),
        ],
        scratch_shapes=[
            pltpu.VMEM((n, d_out), jnp.bfloat16),
            pltpu.VMEM((2, _BM, n), jnp.float32),
            pltpu.SemaphoreType.DMA((2,)),
            pltpu.SemaphoreType.DMA((2,)),
        ],
    )(x, W, b.reshape(1, d_out), adj)
    return out, adj_out
